# baseline (speedup 1.0000x reference)
import functools
import math

import jax
import jax.numpy as jnp
from jax import lax
from jax.experimental import pallas as pl
from jax.experimental.pallas import tpu as pltpu
from jax.experimental.pallas import tpu_sc as plsc

_NC, _NS, _L = 2, 16, 16
_NW = _NC * _NS
_B, _T, _DIM = 4096, 200, 64
_ROWS = _B * _T
_NSPLIT = 4
_QROWS = _ROWS // _NSPLIT
_RPW = _QROWS // _NW
_CHUNK = 128
_NIDX = 128
_NCHUNK = _RPW // _CHUNK
_OUTD = _DIM + 22
_NFRAMES = 2048
_BLK = 1024
_QBLK = _QROWS // _BLK

_CC = (-1.0 / 3628800, 1.0 / 40320, -1.0 / 720, 1.0 / 24, -0.5, 1.0)
_SC = (-1.0 / 39916800, 1.0 / 362880, -1.0 / 5040, 1.0 / 120, -1.0 / 6, 1.0)


def _horner(coefs, z):
    acc = jnp.full((_L,), coefs[0], jnp.float32)
    for c in coefs[1:]:
        acc = acc * z + c
    return acc


def _base_sincos(tf):
    a = tf * (math.pi / 2048.0)
    w = a - (math.pi / 2.0)
    z = w * w
    return _horner(_CC, z), -(w * _horner(_SC, z))


def _body(t_hbm, tab_hbm, out_hbm, t_all, idx0, idx1, out0, out1, tabs, skew,
          gsem0, gsem1, osem0, osem1):
    wid = lax.axis_index("s") * _NC + lax.axis_index("c")
    wbase = wid * _RPW

    @pl.when(lax.axis_index("s") == 0)
    def _():
        pltpu.sync_copy(tab_hbm, tabs)
    plsc.subcore_barrier()

    pltpu.sync_copy(t_hbm.at[pl.ds(wbase, _RPW)], t_all)

    def gathers(idx_b, out_b, gsem):
        return [pltpu.make_async_copy(
            tabs.at[idx_b.at[pl.ds(j * _NIDX, _NIDX)]],
            out_b.at[pl.ds(j * _NIDX, _NIDX)],
            gsem) for j in range(_CHUNK // _NIDX)]

    def stage_a(ci, idx_b, out_b, gsem):
        def idx_group(g, carry):
            tf = t_all[pl.ds(ci * _CHUNK + g * _L, _L)] * 2048.0
            f = tf + 0.5
            i = f.astype(jnp.int32)
            tie = (f == i.astype(jnp.float32)) & ((i & 1) == 1)
            idx_b[pl.ds(g * _L, _L)] = jnp.where(tie, i - 1, i)
            return carry
        lax.fori_loop(0, _CHUNK // _L, idx_group, 0)
        for cp in gathers(idx_b, out_b, gsem):
            cp.start()

    def stage_b(ci, idx_b, out_b, gsem, osem):
        for cp in gathers(idx_b, out_b, gsem):
            cp.wait()
        def four_group(g, carry):
            s, c = _base_sincos(
                t_all[pl.ds(ci * _CHUNK + g * _L, _L)] * 2048.0)
            for j in range(11):
                skew[pl.ds(j * 17, _L)] = s
                skew[pl.ds((11 + j) * 17, _L)] = c
                sc = s * c
                s2 = s * s
                s = sc + sc
                c = 1.0 - (s2 + s2)
            iota17 = lax.iota(jnp.int32, _L) * 17
            for r in range(_L):
                v1 = plsc.load_gather(skew, [iota17 + r])
                v2 = plsc.load_gather(skew, [iota17 + (16 * 17 + r)])
                rr = g * _L + r
                out_b[rr, pl.ds(_DIM, _L)] = v1
                out_b[rr, pl.ds(_DIM + _L, _L)] = v2
            return carry
        lax.fori_loop(0, _CHUNK // _L, four_group, 0)
        pltpu.make_async_copy(
            out_b, out_hbm.at[pl.ds(wbase + ci * _CHUNK, _CHUNK)], osem
        ).start()

    def wait_out(out_b, osem):
        pltpu.make_async_copy(
            out_b, out_hbm.at[pl.ds(wbase, _CHUNK)], osem).wait()

    stage_a(0, idx0, out0, gsem0)
    stage_a(1, idx1, out1, gsem1)
    stage_b(0, idx0, out0, gsem0, osem0)

    def steady(k, carry):
        c = 2 * k
        wait_out(out0, osem0)
        stage_a(c + 2, idx0, out0, gsem0)
        stage_b(c + 1, idx1, out1, gsem1, osem1)
        wait_out(out1, osem1)
        stage_a(c + 3, idx1, out1, gsem1)
        stage_b(c + 2, idx0, out0, gsem0, osem0)
        return carry

    lax.fori_loop(0, (_NCHUNK - 2) // 2, steady, 0)
    stage_b(_NCHUNK - 1, idx1, out1, gsem1, osem1)
    wait_out(out0, osem0)
    wait_out(out1, osem1)


def _tc_first(g_ref, o_ref):
    o_ref[...] = g_ref[:, :_OUTD]


def _tc_next(g_ref, prev_ref, o_ref):
    del prev_ref
    o_ref[...] = g_ref[:, :_OUTD]


@functools.partial(jax.jit, static_argnames=())
def kernel(t, const_embed):
    tflat = t.reshape(_ROWS)
    tab128 = jnp.pad(const_embed, ((0, 0), (0, 128 - _DIM)))
    sc_run = pl.kernel(
        _body,
        out_type=jax.ShapeDtypeStruct((_QROWS, 128), jnp.float32),
        mesh=plsc.VectorSubcoreMesh(core_axis_name="c", subcore_axis_name="s"),
        scratch_types=[
            pltpu.VMEM((_RPW,), jnp.float32),
            pltpu.VMEM((_CHUNK,), jnp.int32),
            pltpu.VMEM((_CHUNK,), jnp.int32),
            pltpu.VMEM((_CHUNK, 128), jnp.float32),
            pltpu.VMEM((_CHUNK, 128), jnp.float32),
            pltpu.VMEM_SHARED((_NFRAMES, 128), jnp.float32),
            pltpu.VMEM((544,), jnp.float32),
            pltpu.SemaphoreType.DMA,
            pltpu.SemaphoreType.DMA,
            pltpu.SemaphoreType.DMA,
            pltpu.SemaphoreType.DMA,
        ],
        compiler_params=pltpu.CompilerParams(needs_layout_passes=False),
    )
    out_sd = jax.ShapeDtypeStruct((_ROWS, _OUTD), jnp.float32)
    out = None
    for q in range(_NSPLIT):
        gq = sc_run(lax.slice(tflat, (q * _QROWS,), ((q + 1) * _QROWS,)),
                    tab128)
        if q == 0:
            out = pl.pallas_call(
                _tc_first,
                grid=(_QBLK,),
                in_specs=[pl.BlockSpec((_BLK, 128), lambda j: (j, 0))],
                out_specs=pl.BlockSpec((_BLK, _OUTD), lambda j: (j, 0)),
                out_shape=out_sd,
            )(gq)
        else:
            out = pl.pallas_call(
                _tc_next,
                grid=(_QBLK,),
                in_specs=[
                    pl.BlockSpec((_BLK, 128), lambda j: (j, 0)),
                    pl.BlockSpec(memory_space=pl.ANY),
                ],
                out_specs=pl.BlockSpec(
                    (_BLK, _OUTD), lambda j, off=q * _QBLK: (off + j, 0)),
                out_shape=out_sd,
                input_output_aliases={1: 0},
            )(gq, out)
    return out.reshape(_B, _T, _OUTD)


# speedup vs baseline: 1.6218x; 1.6218x over previous
import functools
import math

import jax
import jax.numpy as jnp
from jax import lax
from jax.experimental import pallas as pl
from jax.experimental.pallas import tpu as pltpu
from jax.experimental.pallas import tpu_sc as plsc

_NC, _NS, _L = 2, 16, 16
_NW = _NC * _NS
_B, _T, _DIM = 4096, 200, 64
_ROWS = _B * _T
_RPW = _ROWS // _NW
_CHUNK = 256
_NIDX = 128
_NCHUNK = _RPW // _CHUNK
_OUTD = _DIM + 22
_NFRAMES = 2048

_CC = (-1.0 / 3628800, 1.0 / 40320, -1.0 / 720, 1.0 / 24, -0.5, 1.0)
_SC = (-1.0 / 39916800, 1.0 / 362880, -1.0 / 5040, 1.0 / 120, -1.0 / 6, 1.0)


def _horner(coefs, z):
    acc = jnp.full((_L,), coefs[0], jnp.float32)
    for c in coefs[1:]:
        acc = acc * z + c
    return acc


def _base_sincos(tf):
    a = tf * (math.pi / 2048.0)
    w = a - (math.pi / 2.0)
    z = w * w
    return _horner(_CC, z), -(w * _horner(_SC, z))


def _body(t_hbm, tab_hbm, lut_hbm, out_hbm, t_all, idx0, idx1, out0, out1,
          tabs, skew, lut_v, gsem0, gsem1, osem0, osem1):
    wid = lax.axis_index("s") * _NC + lax.axis_index("c")
    wbase = wid * _RPW

    @pl.when(lax.axis_index("s") == 0)
    def _():
        pltpu.sync_copy(tab_hbm, tabs)
    plsc.subcore_barrier()

    pltpu.sync_copy(t_hbm.at[pl.ds(wbase, _RPW)], t_all)
    pltpu.sync_copy(lut_hbm, lut_v)

    def gathers(idx_b, out_b, gsem):
        return [pltpu.make_async_copy(
            tabs.at[idx_b.at[pl.ds(j * _NIDX, _NIDX)]],
            out_b.at[pl.ds(j * _NIDX, _NIDX)],
            gsem) for j in range(_CHUNK // _NIDX)]

    def stage_a(ci, idx_b, out_b, gsem):
        def idx_group(g, carry):
            tf = t_all[pl.ds(ci * _CHUNK + g * _L, _L)] * 2048.0
            f = tf + 0.5
            i = f.astype(jnp.int32)
            tie = (f == i.astype(jnp.float32)) & ((i & 1) == 1)
            idx_b[pl.ds(g * _L, _L)] = jnp.where(tie, i - 1, i)
            return carry
        lax.fori_loop(0, _CHUNK // _L, idx_group, 0)
        for cp in gathers(idx_b, out_b, gsem):
            cp.start()

    def stage_b(ci, idx_b, out_b, gsem, osem):
        for cp in gathers(idx_b, out_b, gsem):
            cp.wait()
        def four_group(g, carry):
            tf = t_all[pl.ds(ci * _CHUNK + g * _L, _L)] * 2048.0
            k16 = idx_b[pl.ds(g * _L, _L)]
            b = (tf - k16.astype(jnp.float32)) * (math.pi / 2048.0)
            a2 = k16 + k16
            st = plsc.load_gather(lut_v, [a2])
            ct = plsc.load_gather(lut_v, [a2 + 1])
            cb = 1.0 - 0.5 * (b * b)
            s = st * cb + ct * b
            c = ct * cb - st * b
            for j in range(11):
                skew[pl.ds(j * 17, _L)] = s
                skew[pl.ds((11 + j) * 17, _L)] = c
                sc = s * c
                s2 = s * s
                s = sc + sc
                c = 1.0 - (s2 + s2)
            iota17 = lax.iota(jnp.int32, _L) * 17
            for r in range(_L):
                v1 = plsc.load_gather(skew, [iota17 + r])
                v2 = plsc.load_gather(skew, [iota17 + (16 * 17 + r)])
                rr = g * _L + r
                out_b[rr, pl.ds(_DIM, _L)] = v1
                out_b[rr, pl.ds(_DIM + _L, _L)] = v2
            return carry
        lax.fori_loop(0, _CHUNK // _L, four_group, 0)
        pltpu.make_async_copy(
            out_b, out_hbm.at[pl.ds(wbase + ci * _CHUNK, _CHUNK)], osem
        ).start()

    def wait_out(out_b, osem):
        pltpu.make_async_copy(
            out_b, out_hbm.at[pl.ds(wbase, _CHUNK)], osem).wait()

    stage_a(0, idx0, out0, gsem0)
    stage_a(1, idx1, out1, gsem1)
    stage_b(0, idx0, out0, gsem0, osem0)

    def steady(k, carry):
        c = 2 * k
        wait_out(out0, osem0)
        stage_a(c + 2, idx0, out0, gsem0)
        stage_b(c + 1, idx1, out1, gsem1, osem1)
        wait_out(out1, osem1)
        stage_a(c + 3, idx1, out1, gsem1)
        stage_b(c + 2, idx0, out0, gsem0, osem0)
        return carry

    lax.fori_loop(0, (_NCHUNK - 2) // 2, steady, 0)
    stage_b(_NCHUNK - 1, idx1, out1, gsem1, osem1)
    wait_out(out0, osem0)
    wait_out(out1, osem1)


@functools.partial(jax.jit, static_argnames=())
def kernel(t, const_embed):
    tflat = t.reshape(_ROWS)
    tab128 = jnp.pad(const_embed, ((0, 0), (0, 128 - _DIM)))
    ang = jnp.arange(_NFRAMES, dtype=jnp.float32) * (math.pi / 2048.0)
    lut = jnp.stack([jnp.sin(ang), jnp.cos(ang)], axis=-1).reshape(-1)
    run = pl.kernel(
        _body,
        out_type=jax.ShapeDtypeStruct((_ROWS, 128), jnp.float32),
        mesh=plsc.VectorSubcoreMesh(core_axis_name="c", subcore_axis_name="s"),
        scratch_types=[
            pltpu.VMEM((_RPW,), jnp.float32),
            pltpu.VMEM((_CHUNK,), jnp.int32),
            pltpu.VMEM((_CHUNK,), jnp.int32),
            pltpu.VMEM((_CHUNK, 128), jnp.float32),
            pltpu.VMEM((_CHUNK, 128), jnp.float32),
            pltpu.VMEM_SHARED((_NFRAMES, 128), jnp.float32),
            pltpu.VMEM((544,), jnp.float32),
            pltpu.VMEM((2 * _NFRAMES,), jnp.float32),
            pltpu.SemaphoreType.DMA,
            pltpu.SemaphoreType.DMA,
            pltpu.SemaphoreType.DMA,
            pltpu.SemaphoreType.DMA,
        ],
        compiler_params=pltpu.CompilerParams(needs_layout_passes=False),
    )
    out = run(tflat, tab128, lut)
    return out[:, :_OUTD].reshape(_B, _T, _OUTD)
